# baseline (speedup 1.0000x reference)
import jax
import jax.numpy as jnp
from jax import lax
from jax.experimental import pallas as pl
from jax.experimental.pallas import tpu as pltpu
from jax.experimental.pallas import tpu_sc as plsc

DIM = 3
PP1 = 4
B = 4096
NCP = 1024
OUT = 512
L = 16

NC = 2
NS = 16
NW = NC * NS
B_PER_W = B // NW
NB = 8
NCHUNK = B_PER_W // NB
NG = OUT // L

ROW = NCP * PP1
OROW = OUT * DIM


def _body(ctrl_hbm, nut_hbm, us_hbm, out_hbm,
          cp0, cp1, ob0, ob1, nut_v, us_v, si0, si1, so0, so1):
    wid = lax.axis_index("s") * NC + lax.axis_index("c")
    base = wid * B_PER_W

    pltpu.sync_copy(nut_hbm, nut_v)
    pltpu.sync_copy(us_hbm, us_v)

    def in_copy(c, buf, sem):
        return pltpu.make_async_copy(
            ctrl_hbm.at[pl.ds((base + c * NB) * ROW, NB * ROW)], buf, sem)

    def out_copy(c, buf, sem):
        return pltpu.make_async_copy(
            buf, out_hbm.at[pl.ds((base + c * NB) * OROW, NB * OROW)], sem)

    in_copy(0, cp0, si0).start()
    in_copy(1, cp1, si1).start()

    def compute(cp, ob):
        def g_body(g, carry):
            off = g * L
            s16 = us_v[pl.ds(off, L)]
            s4 = s16 * 4
            nus = [nut_v[pl.ds(off + k * OUT, L)] for k in range(PP1)]
            fidx = [[s4 + (4 * (k - 3) + d) for d in range(PP1)]
                    for k in range(PP1)]
            u3 = (lax.iota(jnp.int32, L) + off) * 3
            for b in range(NB):
                acc = [None] * PP1
                for d in range(PP1):
                    a = None
                    for k in range(PP1):
                        v = plsc.load_gather(cp, [fidx[k][d] + b * ROW])
                        a = nus[k] * v if a is None else a + nus[k] * v
                    acc[d] = a
                rw = 1.0 / acc[PP1 - 1]
                for d in range(DIM):
                    plsc.store_scatter(ob, [u3 + (d + b * OROW)], acc[d] * rw)
            return carry

        lax.fori_loop(0, NG, g_body, 0)

    def pair_body(p, carry):
        for (i, cp, ob, si, so) in ((p * 2, cp0, ob0, si0, so0),
                                    (p * 2 + 1, cp1, ob1, si1, so1)):
            in_copy(i, cp, si).wait()

            @pl.when(p >= 1)
            def _():
                out_copy(i - 2, ob, so).wait()

            compute(cp, ob)
            out_copy(i, ob, so).start()

            @pl.when(p < NCHUNK // 2 - 1)
            def _():
                in_copy(i + 2, cp, si).start()
        return carry

    lax.fori_loop(0, NCHUNK // 2, pair_body, 0)

    out_copy(NCHUNK - 2, ob0, so0).wait()
    out_copy(NCHUNK - 1, ob1, so1).wait()


def kernel(ctrl_pts, Nu, uspan):
    ctrl_flat = ctrl_pts.reshape(B * ROW)
    nut = Nu.T.reshape(PP1 * OUT)
    us = uspan.astype(jnp.int32)
    run = pl.kernel(
        _body,
        mesh=plsc.VectorSubcoreMesh(core_axis_name="c", subcore_axis_name="s"),
        compiler_params=pltpu.CompilerParams(needs_layout_passes=False),
        out_type=jax.ShapeDtypeStruct((B * OROW,), jnp.float32),
        scratch_types=[
            pltpu.VMEM((NB * ROW,), jnp.float32),
            pltpu.VMEM((NB * ROW,), jnp.float32),
            pltpu.VMEM((NB * OROW,), jnp.float32),
            pltpu.VMEM((NB * OROW,), jnp.float32),
            pltpu.VMEM((PP1 * OUT,), jnp.float32),
            pltpu.VMEM((OUT,), jnp.int32),
            pltpu.SemaphoreType.DMA,
            pltpu.SemaphoreType.DMA,
            pltpu.SemaphoreType.DMA,
            pltpu.SemaphoreType.DMA,
        ],
    )
    out = run(ctrl_flat, nut, us)
    return out.reshape(B, OUT, DIM)


# speedup vs baseline: 71.1551x; 71.1551x over previous
import jax
import jax.numpy as jnp
from jax import lax
from jax.experimental import pallas as pl
from jax.experimental.pallas import tpu as pltpu
from jax.experimental.pallas import tpu_sc as plsc

DIM = 3
PP1 = 4
B = 4096
NCP = 1024
OUT = 512
L = 16

NC = 2
NS = 16
NW = NC * NS
B_PER_W = B // NW
NB = 8
NCHUNK = B_PER_W // NB
NG = OUT // L

ROW = NCP * PP1
OBLK = PP1 * NB * 128


def _body(ctrl_hbm, nut_hbm, us_hbm, out_hbm,
          cp0, cp1, ob0, ob1, nut_v, us_v, si0, si1, so0, so1):
    wid = lax.axis_index("s") * NC + lax.axis_index("c")
    base = wid * B_PER_W

    pltpu.sync_copy(nut_hbm, nut_v)
    pltpu.sync_copy(us_hbm, us_v)

    def in_copy(c, buf, sem):
        return pltpu.make_async_copy(
            ctrl_hbm.at[pl.ds((base + c * NB) * ROW, NB * ROW)], buf, sem)

    def out_copy(c, buf, sem):
        bg = wid * NCHUNK + c
        return [pltpu.make_async_copy(
            buf.at[pl.ds(d * OBLK, OBLK)],
            out_hbm.at[pl.ds(d * (512 * OBLK) + bg * OBLK, OBLK)], sem)
            for d in range(DIM)]

    in_copy(0, cp0, si0).start()
    in_copy(1, cp1, si1).start()

    def compute(cp, ob):
        def g_body(g, carry):
            off = g * L
            s16 = us_v[pl.ds(off, L)]
            nus = [nut_v[pl.ds(off + k * OUT, L)] for k in range(PP1)]
            fidx = []
            for k in range(PP1):
                row = s16 + (k - 3)
                bk = ((row >> 7) << 9) + (row & 127)
                fidx.append([bk + d * 128 for d in range(PP1)])
            odst = ((off >> 7) << 10) + (off & 127)
            for b in range(NB):
                cpb = cp.at[pl.ds(b * ROW, ROW)]
                acc = [None] * PP1
                for d in range(PP1):
                    a = None
                    for k in range(PP1):
                        v = plsc.load_gather(cpb, [fidx[k][d]])
                        a = nus[k] * v if a is None else a + nus[k] * v
                    acc[d] = a
                rw = 1.0 / acc[PP1 - 1]
                for d in range(DIM):
                    ob[pl.ds(odst + (d * OBLK + b * 128), L)] = acc[d] * rw
            return carry

        lax.fori_loop(0, NG, g_body, 0)

    def pair_body(p, carry):
        for (i, cp, ob, si, so) in ((p * 2, cp0, ob0, si0, so0),
                                    (p * 2 + 1, cp1, ob1, si1, so1)):
            in_copy(i, cp, si).wait()

            @pl.when(p >= 1)
            def _():
                for cpy in out_copy(i - 2, ob, so):
                    cpy.wait()

            compute(cp, ob)
            for cpy in out_copy(i, ob, so):
                cpy.start()

            @pl.when(p < NCHUNK // 2 - 1)
            def _():
                in_copy(i + 2, cp, si).start()
        return carry

    lax.fori_loop(0, NCHUNK // 2, pair_body, 0)

    for cpy in out_copy(NCHUNK - 2, ob0, so0):
        cpy.wait()
    for cpy in out_copy(NCHUNK - 1, ob1, so1):
        cpy.wait()


def kernel(ctrl_pts, Nu, uspan):
    ctrl_flat = ctrl_pts.reshape(B, 8, 128, PP1).transpose(0, 1, 3, 2).reshape(
        B * ROW)
    nut = Nu.T.reshape(PP1 * OUT)
    us = uspan.astype(jnp.int32)
    run = pl.kernel(
        _body,
        mesh=plsc.VectorSubcoreMesh(core_axis_name="c", subcore_axis_name="s"),
        compiler_params=pltpu.CompilerParams(needs_layout_passes=False),
        out_type=jax.ShapeDtypeStruct((B * OUT * DIM,), jnp.float32),
        scratch_types=[
            pltpu.VMEM((NB * ROW,), jnp.float32),
            pltpu.VMEM((NB * ROW,), jnp.float32),
            pltpu.VMEM((DIM * OBLK,), jnp.float32),
            pltpu.VMEM((DIM * OBLK,), jnp.float32),
            pltpu.VMEM((PP1 * OUT,), jnp.float32),
            pltpu.VMEM((OUT,), jnp.int32),
            pltpu.SemaphoreType.DMA,
            pltpu.SemaphoreType.DMA,
            pltpu.SemaphoreType.DMA,
            pltpu.SemaphoreType.DMA,
        ],
    )
    out = run(ctrl_flat, nut, us)
    return out.reshape(DIM, 512, PP1, NB, 128).transpose(1, 3, 2, 4, 0).reshape(
        B, OUT, DIM)
